# Initial kernel scaffold; baseline (speedup 1.0000x reference)
import jax
import jax.numpy as jnp
from jax.experimental import pallas as pl


def kernel(x, edge_index, W, b):
    raise NotImplementedError("write your pallas kernel here")


# speedup vs baseline: 15.0678x; 15.0678x over previous
import functools

import jax
import jax.numpy as jnp
from jax import lax
from jax.experimental import pallas as pl
from jax.experimental.pallas import tpu as pltpu
from jax.experimental.pallas import tpu_sc as plsc

NC = 2
NS = 16
NW = NC * NS
CH = 128
ZR = 64


def _zero_vmem_1d(ref, n):
    def body(i, c):
        ref[pl.ds(i * 16, 16)] = jnp.zeros((16,), jnp.float32)
        return c
    lax.fori_loop(0, n // 16, body, 0)


def _zero_vmem_2d(ref, rows):
    def body(i, c):
        r = i // 8
        j = i % 8
        ref[r, pl.ds(j * 16, 16)] = jnp.zeros((16,), jnp.float32)
        return c
    lax.fori_loop(0, rows * 8, body, 0)


def _make_deg_kernel(np_, nch):
    stripe = np_ // NS
    mesh = plsc.VectorSubcoreMesh(core_axis_name="c", subcore_axis_name="s",
                                  num_cores=NC, num_subcores=NS)

    @functools.partial(
        pl.kernel,
        out_type=jax.ShapeDtypeStruct((NC * np_,), jnp.float32),
        mesh=mesh,
        scratch_types=[
            pltpu.VMEM((nch, CH), jnp.int32),
            pltpu.VMEM((CH,), jnp.float32),
            pltpu.VMEM((stripe,), jnp.float32),
            pltpu.VMEM_SHARED((np_,), jnp.float32),
        ],
    )
    def deg_kernel(dst_hbm, degp_hbm, idx_v, ones_v, zb_v, deg_sh):
        c = lax.axis_index("c")
        s = lax.axis_index("s")
        w = c * NS + s

        _zero_vmem_1d(zb_v, stripe)

        def ones_body(i, cc):
            ones_v[pl.ds(i * 16, 16)] = jnp.ones((16,), jnp.float32)
            return cc
        lax.fori_loop(0, CH // 16, ones_body, 0)

        pltpu.sync_copy(zb_v, deg_sh.at[pl.ds(s * stripe, stripe)])
        pltpu.sync_copy(dst_hbm.at[pl.ds(w * nch, nch), :], idx_v)
        plsc.subcore_barrier()

        def body(j, cc):
            pltpu.sync_copy(ones_v, deg_sh.at[idx_v.at[j]], add=True)
            return cc
        lax.fori_loop(0, nch, body, 0)

        plsc.subcore_barrier()
        pltpu.sync_copy(deg_sh.at[pl.ds(s * stripe, stripe)],
                        degp_hbm.at[pl.ds(c * np_ + s * stripe, stripe)])

    return deg_kernel


def _make_agg_kernel(np_, nch, d):
    stripe = np_ // NS
    mesh = plsc.VectorSubcoreMesh(core_axis_name="c", subcore_axis_name="s",
                                  num_cores=NC, num_subcores=NS)

    @functools.partial(
        pl.kernel,
        out_type=jax.ShapeDtypeStruct((NC, np_, d), jnp.float32),
        mesh=mesh,
        scratch_types=[
            pltpu.VMEM((nch, CH), jnp.int32),
            pltpu.VMEM((nch, CH), jnp.int32),
            pltpu.VMEM((CH, d), jnp.float32),
            pltpu.VMEM((ZR, d), jnp.float32),
            pltpu.VMEM_SHARED((np_, d), jnp.float32),
            pltpu.SemaphoreType.DMA,
        ],
    )
    def agg_kernel(g_hbm, src_hbm, dst_hbm, accp_hbm,
                   sidx, didx, rows, zb, acc_sh, sem):
        c = lax.axis_index("c")
        s = lax.axis_index("s")
        w = c * NS + s

        _zero_vmem_2d(zb, ZR)

        def zbody(i, cc):
            pltpu.sync_copy(zb, acc_sh.at[pl.ds(s * stripe + i * ZR, ZR), :])
            return cc
        lax.fori_loop(0, stripe // ZR, zbody, 0)

        pltpu.sync_copy(src_hbm.at[pl.ds(w * nch, nch), :], sidx)
        pltpu.sync_copy(dst_hbm.at[pl.ds(w * nch, nch), :], didx)
        plsc.subcore_barrier()

        def body(j, cc):
            pltpu.async_copy(g_hbm.at[sidx.at[j]], rows, sem).wait()
            pltpu.sync_copy(rows, acc_sh.at[didx.at[j]], add=True)
            return cc
        lax.fori_loop(0, nch, body, 0)

        plsc.subcore_barrier()
        pltpu.sync_copy(acc_sh.at[pl.ds(s * stripe, stripe), :],
                        accp_hbm.at[c, pl.ds(s * stripe, stripe), :])

    return agg_kernel


def _mm_body(x_ref, w_ref, degp_ref, g_ref):
    deg = degp_ref[0, :] + degp_ref[1, :] + 1.0
    dinv = lax.rsqrt(deg)
    h = jnp.dot(x_ref[:, :], w_ref[:, :], preferred_element_type=jnp.float32)
    g_ref[:, :] = h * dinv[:, None]


def _out_body(acc_ref, g_ref, degp_ref, b_ref, o_ref):
    deg = degp_ref[0, :] + degp_ref[1, :] + 1.0
    dinv = lax.rsqrt(deg)
    t = (acc_ref[0, :, :] + acc_ref[1, :, :] + g_ref[:, :]) * dinv[:, None]
    t = t + b_ref[:, :]
    m = jnp.max(t, axis=1, keepdims=True)
    lse = jnp.log(jnp.sum(jnp.exp(t - m), axis=1, keepdims=True)) + m
    o_ref[:, :] = t - lse


def kernel(x, edge_index, W, b):
    n, d_in = x.shape
    d = W.shape[1]
    e = edge_index.shape[1]

    np_ = ((n + 2048) // 2048) * 2048
    epw = (((e + NW - 1) // NW) + 8 * CH - 1) // (8 * CH) * (8 * CH)
    nch = epw // CH
    ep = epw * NW
    dummy = np_ - 1

    src = jnp.concatenate(
        [edge_index[0], jnp.full((ep - e,), dummy, jnp.int32)]).reshape(-1, CH)
    dst = jnp.concatenate(
        [edge_index[1], jnp.full((ep - e,), dummy, jnp.int32)]).reshape(-1, CH)
    x_pad = jnp.pad(x, ((0, np_ - n), (0, 0)))
    b2 = b.reshape(1, d)

    degp = _make_deg_kernel(np_, nch)(dst).reshape(NC, np_)

    bm = 512
    g = pl.pallas_call(
        _mm_body,
        grid=(np_ // bm,),
        in_specs=[
            pl.BlockSpec((bm, d_in), lambda i: (i, 0)),
            pl.BlockSpec((d_in, d), lambda i: (0, 0)),
            pl.BlockSpec((NC, bm), lambda i: (0, i)),
        ],
        out_specs=pl.BlockSpec((bm, d), lambda i: (i, 0)),
        out_shape=jax.ShapeDtypeStruct((np_, d), jnp.float32),
    )(x_pad, W, degp)

    accp = _make_agg_kernel(np_, nch, d)(g, src, dst)

    bo = 512
    out = pl.pallas_call(
        _out_body,
        grid=(np_ // bo,),
        in_specs=[
            pl.BlockSpec((NC, bo, d), lambda i: (0, i, 0)),
            pl.BlockSpec((bo, d), lambda i: (i, 0)),
            pl.BlockSpec((NC, bo), lambda i: (0, i)),
            pl.BlockSpec((1, d), lambda i: (0, 0)),
        ],
        out_specs=pl.BlockSpec((bo, d), lambda i: (i, 0)),
        out_shape=jax.ShapeDtypeStruct((np_, d), jnp.float32),
    )(accp, g, degp, b2)

    return out[:n]


# speedup vs baseline: 17.1975x; 1.1413x over previous
import functools

import jax
import jax.numpy as jnp
from jax import lax
from jax.experimental import pallas as pl
from jax.experimental.pallas import tpu as pltpu
from jax.experimental.pallas import tpu_sc as plsc

NC = 2
NS = 16
NW = NC * NS
CH = 128
ZR = 64


def _zero_vmem_1d(ref, n):
    def body(i, c):
        ref[pl.ds(i * 16, 16)] = jnp.zeros((16,), jnp.float32)
        return c
    lax.fori_loop(0, n // 16, body, 0)


def _zero_vmem_2d(ref, rows):
    def body(i, c):
        r = i // 8
        j = i % 8
        ref[r, pl.ds(j * 16, 16)] = jnp.zeros((16,), jnp.float32)
        return c
    lax.fori_loop(0, rows * 8, body, 0)


def _make_deg_kernel(np_, nch):
    stripe = np_ // NS
    mesh = plsc.VectorSubcoreMesh(core_axis_name="c", subcore_axis_name="s",
                                  num_cores=NC, num_subcores=NS)

    @functools.partial(
        pl.kernel,
        out_type=jax.ShapeDtypeStruct((NC * np_,), jnp.float32),
        mesh=mesh,
        scratch_types=[
            pltpu.VMEM((nch, CH), jnp.int32),
            pltpu.VMEM((CH,), jnp.float32),
            pltpu.VMEM((stripe,), jnp.float32),
            pltpu.VMEM_SHARED((np_,), jnp.float32),
        ],
    )
    def deg_kernel(dst_hbm, degp_hbm, idx_v, ones_v, zb_v, deg_sh):
        c = lax.axis_index("c")
        s = lax.axis_index("s")
        w = c * NS + s

        _zero_vmem_1d(zb_v, stripe)

        def ones_body(i, cc):
            ones_v[pl.ds(i * 16, 16)] = jnp.ones((16,), jnp.float32)
            return cc
        lax.fori_loop(0, CH // 16, ones_body, 0)

        pltpu.sync_copy(zb_v, deg_sh.at[pl.ds(s * stripe, stripe)])
        pltpu.sync_copy(dst_hbm.at[pl.ds(w * nch, nch), :], idx_v)
        plsc.subcore_barrier()

        def body(j, cc):
            pltpu.sync_copy(ones_v, deg_sh.at[idx_v.at[j]], add=True)
            return cc
        lax.fori_loop(0, nch, body, 0)

        plsc.subcore_barrier()
        pltpu.sync_copy(deg_sh.at[pl.ds(s * stripe, stripe)],
                        degp_hbm.at[pl.ds(c * np_ + s * stripe, stripe)])

    return deg_kernel


def _make_agg_kernel(np_, nch, d):
    stripe = np_ // NS
    mesh = plsc.VectorSubcoreMesh(core_axis_name="c", subcore_axis_name="s",
                                  num_cores=NC, num_subcores=NS)

    @functools.partial(
        pl.kernel,
        out_type=jax.ShapeDtypeStruct((NC, np_, d), jnp.float32),
        mesh=mesh,
        scratch_types=[
            pltpu.VMEM((nch // 2, CH), jnp.int32),
            pltpu.VMEM((nch // 2, CH), jnp.int32),
            pltpu.VMEM((CH, d), jnp.float32),
            pltpu.VMEM((CH, d), jnp.float32),
            pltpu.VMEM_SHARED((np_, d), jnp.float32),
            pltpu.SemaphoreType.DMA,
            pltpu.SemaphoreType.DMA,
        ],
    )
    def agg_kernel(g_hbm, src_hbm, dst_hbm, accp_hbm,
                   sidx, didx, r0, r1, acc_sh, s0, s1):
        c = lax.axis_index("c")
        s = lax.axis_index("s")
        w = c * NS + s
        slots = ((r0, s0), (r1, s1))
        hch = nch // 2

        _zero_vmem_2d(r0, CH)

        def zbody(i, cc):
            pltpu.sync_copy(r0, acc_sh.at[pl.ds(s * stripe + i * CH, CH), :])
            return cc
        lax.fori_loop(0, stripe // CH, zbody, 0)
        plsc.subcore_barrier()

        for h in range(2):
            base = (w * 2 + h) * hch
            pltpu.sync_copy(src_hbm.at[pl.ds(base, hch), :], sidx)
            pltpu.sync_copy(dst_hbm.at[pl.ds(base, hch), :], didx)

            for bi, (rbuf, sm) in enumerate(slots):
                pltpu.async_copy(g_hbm.at[sidx.at[bi]], rbuf, sm)

            def body(o, cc):
                for bi, (rbuf, sm) in enumerate(slots):
                    j = o * 2 + bi
                    pltpu.make_async_copy(g_hbm.at[sidx.at[j]], rbuf, sm).wait()
                    pltpu.sync_copy(rbuf, acc_sh.at[didx.at[j]], add=True)
                    nj = j + 2

                    @pl.when(nj < hch)
                    def _():
                        pltpu.async_copy(g_hbm.at[sidx.at[nj]], rbuf, sm)
                return cc
            lax.fori_loop(0, hch // 2, body, 0)

        plsc.subcore_barrier()
        pltpu.sync_copy(acc_sh.at[pl.ds(s * stripe, stripe), :],
                        accp_hbm.at[c, pl.ds(s * stripe, stripe), :])

    return agg_kernel


def _mm_body(x_ref, w_ref, degp_ref, g_ref):
    deg = degp_ref[0, :] + degp_ref[1, :] + 1.0
    dinv = lax.rsqrt(deg)
    h = jnp.dot(x_ref[:, :], w_ref[:, :], preferred_element_type=jnp.float32)
    g_ref[:, :] = h * dinv[:, None]


def _out_body(acc_ref, g_ref, degp_ref, b_ref, o_ref):
    deg = degp_ref[0, :] + degp_ref[1, :] + 1.0
    dinv = lax.rsqrt(deg)
    t = (acc_ref[0, :, :] + acc_ref[1, :, :] + g_ref[:, :]) * dinv[:, None]
    t = t + b_ref[:, :]
    m = jnp.max(t, axis=1, keepdims=True)
    lse = jnp.log(jnp.sum(jnp.exp(t - m), axis=1, keepdims=True)) + m
    o_ref[:, :] = t - lse


def kernel(x, edge_index, W, b):
    n, d_in = x.shape
    d = W.shape[1]
    e = edge_index.shape[1]

    np_ = ((n + 2048) // 2048) * 2048
    epw = (((e + NW - 1) // NW) + 8 * CH - 1) // (8 * CH) * (8 * CH)
    nch = epw // CH
    ep = epw * NW
    dummy = np_ - 1

    src = jnp.concatenate(
        [edge_index[0], jnp.full((ep - e,), dummy, jnp.int32)]).reshape(-1, CH)
    dst = jnp.concatenate(
        [edge_index[1], jnp.full((ep - e,), dummy, jnp.int32)]).reshape(-1, CH)
    x_pad = jnp.pad(x, ((0, np_ - n), (0, 0)))
    b2 = b.reshape(1, d)

    degp = _make_deg_kernel(np_, nch)(dst).reshape(NC, np_)

    bm = 512
    g = pl.pallas_call(
        _mm_body,
        grid=(np_ // bm,),
        in_specs=[
            pl.BlockSpec((bm, d_in), lambda i: (i, 0)),
            pl.BlockSpec((d_in, d), lambda i: (0, 0)),
            pl.BlockSpec((NC, bm), lambda i: (0, i)),
        ],
        out_specs=pl.BlockSpec((bm, d), lambda i: (i, 0)),
        out_shape=jax.ShapeDtypeStruct((np_, d), jnp.float32),
    )(x_pad, W, degp)

    accp = _make_agg_kernel(np_, nch, d)(g, src, dst)

    bo = 512
    out = pl.pallas_call(
        _out_body,
        grid=(np_ // bo,),
        in_specs=[
            pl.BlockSpec((NC, bo, d), lambda i: (0, i, 0)),
            pl.BlockSpec((bo, d), lambda i: (i, 0)),
            pl.BlockSpec((NC, bo), lambda i: (0, i)),
            pl.BlockSpec((1, d), lambda i: (0, 0)),
        ],
        out_specs=pl.BlockSpec((bo, d), lambda i: (i, 0)),
        out_shape=jax.ShapeDtypeStruct((np_, d), jnp.float32),
    )(accp, g, degp, b2)

    return out[:n]
